# baseline (speedup 1.0000x reference)
import functools

import jax
import jax.numpy as jnp
from jax import lax
from jax.experimental import pallas as pl
from jax.experimental.pallas import tpu as pltpu
from jax.experimental.pallas import tpu_sc as plsc

_N = 16384
_C = 1000
_T = 8192
_BLK = 2048
_NW = 32
_RPW = (_N - _T) // _NW
_CH = 32
_NCHUNK = _RPW // _CH
_YPW = _N // _NW


def _sc_body(out_hbm, y_hbm, sp_hbm, pick_hbm, hist_hbm,
             y_v, yh_v, ones_v, buf_a, buf_b, acc_v, pick_v, zeros_v,
             hist_sh, sem_a, sem_b):
    cid = lax.axis_index("c")
    sid = lax.axis_index("s")
    wid = sid * 2 + cid
    base = wid * _RPW

    lane = lax.broadcasted_iota(jnp.int32, (16,), 0)
    tail_mask = lane >= 8
    zeros16 = jnp.zeros((16,), jnp.float32)
    ones16i = jnp.ones((16,), jnp.int32)

    pltpu.sync_copy(y_hbm.at[pl.ds(_T + base, _RPW)], y_v)
    pltpu.sync_copy(y_hbm.at[pl.ds(wid * _YPW, _YPW)], yh_v)

    def _init(i, _):
        ones_v[pl.ds(i * 16, 16)] = ones16i
        return 0
    lax.fori_loop(0, _YPW // 16, _init, 0)

    def _zero(i, _):
        zeros_v[pl.ds(i * 16, 16)] = jnp.zeros((16,), jnp.int32)
        return 0
    lax.fori_loop(0, 1024 // 16, _zero, 0)

    @pl.when(sid == 0)
    def _():
        pltpu.sync_copy(zeros_v, hist_sh)
    plsc.subcore_barrier()
    pltpu.sync_copy(ones_v, hist_sh.at[yh_v], add=True)
    plsc.subcore_barrier()

    def _issue(ck, buf, sem):
        pltpu.async_copy(out_hbm.at[pl.ds(base + ck * _CH, _CH)], buf, sem)

    def _drain(buf, sem):
        pltpu.make_async_copy(out_hbm.at[pl.ds(base, _CH)], buf, sem).wait()

    def _process(ck, buf):
        def _row(r, _):
            rt = ck * _CH + r
            ys = y_v[pl.ds(ck * _CH + (r // 16) * 16, 16)]
            idxv = jnp.zeros((16,), jnp.int32) + (r % 16)
            y_spl = lax.gather(
                ys, idxv[:, None],
                lax.GatherDimensionNumbers(
                    offset_dims=(), collapsed_slice_dims=(0,),
                    start_index_map=(0,)),
                slice_sizes=(1,),
                mode=lax.GatherScatterMode.PROMISE_IN_BOUNDS)
            diff = y_spl - lane
            q = jnp.where((diff & 15) == 0, diff >> 4,
                          jnp.full((16,), -1, jnp.int32))
            total = zeros16
            pick = zeros16
            for j in range(_C // 16):
                v = buf[r, pl.ds(j * 16, 16)]
                total = total + v
                pick = pick + jnp.where(q == j, v, zeros16)
            tail = buf[r, pl.ds(_C - 16, 16)]
            total = total + jnp.where(tail_mask, tail, zeros16)
            pick = pick + jnp.where((diff == 984) & tail_mask, tail, zeros16)
            acc_v[rt // 8, pl.ds(16 * (rt % 8), 16)] = total
            pick_v[rt // 8, pl.ds(16 * (rt % 8), 16)] = pick
            return 0
        lax.fori_loop(0, _CH, _row, 0)

    _issue(0, buf_a, sem_a)
    _issue(1, buf_b, sem_b)

    def _outer(k2, _):
        for b, buf, sem in ((0, buf_a, sem_a), (1, buf_b, sem_b)):
            ck = 2 * k2 + b
            _drain(buf, sem)
            _process(ck, buf)
            @pl.when(ck + 2 < _NCHUNK)
            def _():
                _issue(ck + 2, buf, sem)
        return 0
    lax.fori_loop(0, _NCHUNK // 2, _outer, 0)
    if _NCHUNK % 2:
        _drain(buf_a, sem_a)
        _process(_NCHUNK - 1, buf_a)

    nrow = (16 * _RPW) // 128
    pltpu.sync_copy(acc_v, sp_hbm.at[pl.ds(wid * nrow, nrow)])
    pltpu.sync_copy(pick_v, pick_hbm.at[pl.ds(wid * nrow, nrow)])
    @pl.when(sid == 0)
    def _():
        pltpu.sync_copy(hist_sh, hist_hbm.at[pl.ds(cid * 1024, 1024)])


_sc_pass = functools.partial(
    pl.kernel,
    out_type=[
        jax.ShapeDtypeStruct((16 * (_N - _T) // 128, 128), jnp.float32),
        jax.ShapeDtypeStruct((16 * (_N - _T) // 128, 128), jnp.float32),
        jax.ShapeDtypeStruct((2 * 1024,), jnp.int32),
    ],
    mesh=plsc.VectorSubcoreMesh(core_axis_name="c", subcore_axis_name="s"),
    scratch_types=[
        pltpu.VMEM((_RPW,), jnp.int32),
        pltpu.VMEM((_YPW,), jnp.int32),
        pltpu.VMEM((_YPW,), jnp.int32),
        pltpu.VMEM((_CH, _C), jnp.float32),
        pltpu.VMEM((_CH, _C), jnp.float32),
        pltpu.VMEM((16 * _RPW // 128, 128), jnp.float32),
        pltpu.VMEM((16 * _RPW // 128, 128), jnp.float32),
        pltpu.VMEM((1024,), jnp.int32),
        pltpu.VMEM_SHARED((1024,), jnp.int32),
        pltpu.SemaphoreType.DMA,
        pltpu.SemaphoreType.DMA,
    ],
)(_sc_body)


def _tc1_body(out_ref, y_ref, acc_ref):
    i = pl.program_id(0)
    blk, C = out_ref.shape

    @pl.when(i == 0)
    def _init():
        acc_ref[0, 0] = jnp.float32(0.0)

    x = out_ref[...]
    yv = y_ref[...]
    col = lax.broadcasted_iota(jnp.int32, (blk, C), 1)
    onehot = col == yv
    s = jnp.sum(x, axis=1, keepdims=True)
    picked = jnp.sum(jnp.where(onehot, x, 0.0), axis=1, keepdims=True)
    acc_ref[0, 0] += jnp.sum(jnp.log(s) - jnp.log(picked))


def _combine_body(sp_ref, pick_ref, hist_ref, acc_ref, loss_ref):
    col = lax.broadcasted_iota(jnp.int32, (128, 8), 0)
    grp = lax.broadcasted_iota(jnp.int32, (128, 8), 1)
    m = (col // 16 == grp).astype(jnp.float32)
    dims = (((1,), (0,)), ((), ()))
    s8 = jax.lax.dot_general(sp_ref[...], m, dims,
                             preferred_element_type=jnp.float32)
    p8 = jax.lax.dot_general(pick_ref[...], m, dims,
                             preferred_element_type=jnp.float32)
    slog_s = jnp.sum(jnp.log(s8))
    slog_p = jnp.sum(jnp.log(p8))
    hist = hist_ref[...].reshape(2, 8, 128).astype(jnp.float32)
    cnt = jnp.sum(hist, axis=0)
    cterm = jnp.sum(cnt * jnp.log(jnp.maximum(cnt, 1.0)))
    nf = jnp.float32(_N)
    loss_ref[0, 0] = (jnp.log(nf)
                      + (acc_ref[0, 0] + slog_s - slog_p - cterm) / nf)


@jax.jit
def _arb_loss(output, y):
    y = y.astype(jnp.int32)
    acc = pl.pallas_call(
        _tc1_body,
        grid=(_T // _BLK,),
        in_specs=[
            pl.BlockSpec((_BLK, _C), lambda i: (i, 0)),
            pl.BlockSpec((_BLK, 1), lambda i: (i, 0)),
        ],
        out_specs=pl.BlockSpec(memory_space=pltpu.SMEM),
        out_shape=jax.ShapeDtypeStruct((1, 1), jnp.float32),
        compiler_params=pltpu.CompilerParams(
            dimension_semantics=("arbitrary",),
        ),
    )(output, y.reshape(_N, 1))
    sp, pick, hist = _sc_pass(output[_T:], y)
    out = pl.pallas_call(
        _combine_body,
        out_specs=pl.BlockSpec(memory_space=pltpu.SMEM),
        out_shape=jax.ShapeDtypeStruct((1, 1), jnp.float32),
        in_specs=[
            pl.BlockSpec((16 * (_N - _T) // 128, 128), lambda: (0, 0)),
            pl.BlockSpec((16 * (_N - _T) // 128, 128), lambda: (0, 0)),
            pl.BlockSpec((16, 128), lambda: (0, 0)),
            pl.BlockSpec(memory_space=pltpu.SMEM),
        ],
    )(sp, pick, hist.reshape(16, 128), acc)
    return out.reshape(())


def kernel(output, y):
    return _arb_loss(output, y)


# speedup vs baseline: 1.2582x; 1.2582x over previous
import functools

import jax
import jax.numpy as jnp
from jax import lax
from jax.experimental import pallas as pl
from jax.experimental.pallas import tpu as pltpu
from jax.experimental.pallas import tpu_sc as plsc

_N = 16384
_C = 1000
_BLK = 2048
_NW = 32
_YPW = _N // _NW


def _sc_hist_body(y_hbm, hist_hbm, yh_v, ones_v, zeros_v, hist_sh):
    cid = lax.axis_index("c")
    sid = lax.axis_index("s")
    wid = sid * 2 + cid

    ones16i = jnp.ones((16,), jnp.int32)
    pltpu.sync_copy(y_hbm.at[pl.ds(wid * _YPW, _YPW)], yh_v)

    def _init(i, _):
        ones_v[pl.ds(i * 16, 16)] = ones16i
        return 0
    lax.fori_loop(0, _YPW // 16, _init, 0)

    def _zero(i, _):
        zeros_v[pl.ds(i * 16, 16)] = jnp.zeros((16,), jnp.int32)
        return 0
    lax.fori_loop(0, 1024 // 16, _zero, 0)

    @pl.when(sid == 0)
    def _():
        pltpu.sync_copy(zeros_v, hist_sh)
    plsc.subcore_barrier()
    pltpu.sync_copy(ones_v, hist_sh.at[yh_v], add=True)
    plsc.subcore_barrier()

    @pl.when(sid == 0)
    def _():
        pltpu.sync_copy(hist_sh, hist_hbm.at[pl.ds(cid * 1024, 1024)])


_sc_hist = functools.partial(
    pl.kernel,
    out_type=[jax.ShapeDtypeStruct((2 * 1024,), jnp.int32)],
    mesh=plsc.VectorSubcoreMesh(core_axis_name="c", subcore_axis_name="s"),
    scratch_types=[
        pltpu.VMEM((_YPW,), jnp.int32),
        pltpu.VMEM((_YPW,), jnp.int32),
        pltpu.VMEM((1024,), jnp.int32),
        pltpu.VMEM_SHARED((1024,), jnp.int32),
    ],
)(_sc_hist_body)


def _tc1_body(out_ref, y_ref, acc_ref):
    i = pl.program_id(0)
    blk, C = out_ref.shape

    @pl.when(i == 0)
    def _init():
        acc_ref[0, 0] = jnp.float32(0.0)

    x = out_ref[...]
    yv = y_ref[...]
    col = lax.broadcasted_iota(jnp.int32, (blk, C), 1)
    onehot = col == yv
    s = jnp.sum(x, axis=1, keepdims=True)
    picked = jnp.sum(jnp.where(onehot, x, 0.0), axis=1, keepdims=True)
    acc_ref[0, 0] += jnp.sum(jnp.log(s) - jnp.log(picked))


def _combine_body(hist_ref, acc_ref, loss_ref):
    hist = hist_ref[...].reshape(2, 8, 128).astype(jnp.float32)
    cnt = jnp.sum(hist, axis=0)
    cterm = jnp.sum(cnt * jnp.log(jnp.maximum(cnt, 1.0)))
    nf = jnp.float32(_N)
    loss_ref[0, 0] = jnp.log(nf) + (acc_ref[0, 0] - cterm) / nf


@jax.jit
def _arb_loss(output, y):
    y = y.astype(jnp.int32)
    (hist,) = _sc_hist(y)
    acc = pl.pallas_call(
        _tc1_body,
        grid=(_N // _BLK,),
        in_specs=[
            pl.BlockSpec((_BLK, _C), lambda i: (i, 0)),
            pl.BlockSpec((_BLK, 1), lambda i: (i, 0)),
        ],
        out_specs=pl.BlockSpec(memory_space=pltpu.SMEM),
        out_shape=jax.ShapeDtypeStruct((1, 1), jnp.float32),
        compiler_params=pltpu.CompilerParams(
            dimension_semantics=("arbitrary",),
        ),
    )(output, y.reshape(_N, 1))
    out = pl.pallas_call(
        _combine_body,
        out_specs=pl.BlockSpec(memory_space=pltpu.SMEM),
        out_shape=jax.ShapeDtypeStruct((1, 1), jnp.float32),
        in_specs=[
            pl.BlockSpec((16, 128), lambda: (0, 0)),
            pl.BlockSpec(memory_space=pltpu.SMEM),
        ],
    )(hist.reshape(16, 128), acc)
    return out.reshape(())


def kernel(output, y):
    return _arb_loss(output, y)


# speedup vs baseline: 3.2558x; 2.5876x over previous
import functools

import jax
import jax.numpy as jnp
from jax import lax
from jax.experimental import pallas as pl
from jax.experimental.pallas import tpu as pltpu
from jax.experimental.pallas import tpu_sc as plsc

_N = 16384
_C = 1000
_BLK = 2048
_NW = 32
_YPW = _N // _NW


def _sc_hist_body(y_hbm, hist_hbm, yh_v, ones_v, zeros_v, hist_sh):
    cid = lax.axis_index("c")
    sid = lax.axis_index("s")
    wid = sid * 2 + cid

    ones16i = jnp.ones((16,), jnp.int32)
    pltpu.sync_copy(y_hbm.at[pl.ds(wid * _YPW, _YPW)], yh_v)

    def _init(i, _):
        ones_v[pl.ds(i * 16, 16)] = ones16i
        return 0
    lax.fori_loop(0, _YPW // 16, _init, 0)

    def _zero(i, _):
        zeros_v[pl.ds(i * 16, 16)] = jnp.zeros((16,), jnp.int32)
        return 0
    lax.fori_loop(0, 1024 // 16, _zero, 0)

    @pl.when(sid == 0)
    def _():
        pltpu.sync_copy(zeros_v, hist_sh)
    plsc.subcore_barrier()
    pltpu.sync_copy(ones_v, hist_sh.at[yh_v], add=True)
    plsc.subcore_barrier()

    @pl.when(sid == 0)
    def _():
        pltpu.sync_copy(hist_sh, hist_hbm.at[pl.ds(cid * 1024, 1024)])


_sc_hist = functools.partial(
    pl.kernel,
    out_type=[jax.ShapeDtypeStruct((2 * 1024,), jnp.int32)],
    mesh=plsc.VectorSubcoreMesh(core_axis_name="c", subcore_axis_name="s"),
    scratch_types=[
        pltpu.VMEM((_YPW,), jnp.int32),
        pltpu.VMEM((_YPW,), jnp.int32),
        pltpu.VMEM((1024,), jnp.int32),
        pltpu.VMEM_SHARED((1024,), jnp.int32),
    ],
)(_sc_hist_body)


def _tc1_body(out_ref, y_ref, acc_ref):
    i = pl.program_id(0)
    C, blk = out_ref.shape

    @pl.when(i == 0)
    def _init():
        acc_ref[0, 0] = jnp.float32(0.0)

    x = out_ref[...]
    yv = y_ref[0, ...]
    row = lax.broadcasted_iota(jnp.int32, (C, blk), 0)
    onehot = row == yv
    s = jnp.sum(x, axis=0, keepdims=True)
    picked = jnp.sum(jnp.where(onehot, x, 0.0), axis=0, keepdims=True)
    acc_ref[0, 0] += jnp.sum(jnp.log(s) - jnp.log(picked))


def _combine_body(hist_ref, acc_ref, loss_ref):
    hist = hist_ref[...].reshape(2, 8, 128).astype(jnp.float32)
    cnt = jnp.sum(hist, axis=0)
    cterm = jnp.sum(cnt * jnp.log(jnp.maximum(cnt, 1.0)))
    nf = jnp.float32(_N)
    loss_ref[0, 0] = jnp.log(nf) + (acc_ref[0, 0] - cterm) / nf


@jax.jit
def _arb_loss(output, y):
    y = y.astype(jnp.int32)
    (hist,) = _sc_hist(y)
    acc = pl.pallas_call(
        _tc1_body,
        grid=(_N // _BLK,),
        in_specs=[
            pl.BlockSpec((_C, _BLK), lambda i: (0, i)),
            pl.BlockSpec((1, 1, _BLK), lambda i: (i, 0, 0)),
        ],
        out_specs=pl.BlockSpec(memory_space=pltpu.SMEM),
        out_shape=jax.ShapeDtypeStruct((1, 1), jnp.float32),
        compiler_params=pltpu.CompilerParams(
            dimension_semantics=("arbitrary",),
        ),
    )(output.T, y.reshape(_N // _BLK, 1, _BLK))
    out = pl.pallas_call(
        _combine_body,
        out_specs=pl.BlockSpec(memory_space=pltpu.SMEM),
        out_shape=jax.ShapeDtypeStruct((1, 1), jnp.float32),
        in_specs=[
            pl.BlockSpec((16, 128), lambda: (0, 0)),
            pl.BlockSpec(memory_space=pltpu.SMEM),
        ],
    )(hist.reshape(16, 128), acc)
    return out.reshape(())


def kernel(output, y):
    return _arb_loss(output, y)
